# baseline (speedup 1.0000x reference)
import functools

import jax
import jax.numpy as jnp
from jax import lax
from jax.experimental import pallas as pl
from jax.experimental.pallas import tpu as pltpu
from jax.experimental.pallas import tpu_sc as plsc

N = 10000
E = 320000
D = 128

NC = 2
NS = 16
NW = NC * NS

EPT = E // NW
C = 40
NCHUNK = EPT // C
RPT = 624
TAIL = N - NS * RPT


def _sc_aggregate(h, src, dst2, attr, emb):
    mesh = plsc.VectorSubcoreMesh(core_axis_name="c", subcore_axis_name="s")

    @functools.partial(
        pl.kernel,
        mesh=mesh,
        out_type=jax.ShapeDtypeStruct((NC, N, D), jnp.float32),
        scratch_types=[
            pltpu.VMEM((2, C), jnp.int32),
            pltpu.VMEM((4, C), jnp.int32),
            pltpu.VMEM((4, C), jnp.int32),
            pltpu.VMEM((8, D), jnp.float32),
            pltpu.VMEM((2, C, D), jnp.float32),
            pltpu.VMEM_SHARED((N, D), jnp.float32),
            pltpu.SemaphoreType.DMA,
            pltpu.SemaphoreType.DMA,
            pltpu.SemaphoreType.DMA,
        ],
    )
    def agg(h_hbm, src_hbm, dst_hbm, attr_hbm, emb_hbm, out_hbm,
            sidx, didx, attrs, embt, hrow, accum,
            isem, hsem, ssem):
        c = lax.axis_index("c")
        s = lax.axis_index("s")
        base = (c * NS + s) * EPT

        pltpu.sync_copy(emb_hbm, embt)

        zero = jnp.zeros((16,), jnp.float32)

        def zb_body(i, carry):
            for b in range(2):
                for j in range(8):
                    hrow[b, i, pl.ds(j * 16, 16)] = zero
            return carry

        lax.fori_loop(0, C, zb_body, 0)
        for k in range(RPT // C):
            pltpu.sync_copy(hrow.at[0],
                            accum.at[pl.ds(s * RPT + k * C, C)])
        pltpu.sync_copy(hrow.at[0, pl.ds(0, RPT - (RPT // C) * C)],
                        accum.at[pl.ds(s * RPT + (RPT // C) * C,
                                       RPT - (RPT // C) * C)])

        @pl.when(s == NS - 1)
        def _zero_tail():
            pltpu.sync_copy(hrow.at[0, pl.ds(0, TAIL)],
                            accum.at[pl.ds(NS * RPT, TAIL)])

        def load_idx(chunk, ib, db):
            off = base + chunk * C
            pltpu.async_copy(src_hbm.at[pl.ds(off, C)], sidx.at[ib], isem)
            pltpu.async_copy(attr_hbm.at[pl.ds(off, C)], attrs.at[db], isem)
            pltpu.async_copy(dst_hbm.at[pl.ds(off, C)], didx.at[db], isem)

        def wait_idx():
            for _ in range(2):
                pltpu.make_async_copy(
                    src_hbm.at[pl.ds(0, C)], sidx.at[0], isem).wait()
            pltpu.make_async_copy(
                attr_hbm.at[pl.ds(0, C)], attrs.at[0], isem).wait()

        load_idx(0, 0, 0)
        load_idx(1, 1, 1)
        wait_idx()
        wait_idx()
        pltpu.async_copy(h_hbm.at[sidx.at[0]], hrow.at[0], hsem)

        plsc.subcore_barrier()

        def chunk_body(i, carry):
            p = lax.rem(i, 2)
            q = lax.rem(i + 1, 2)
            m = lax.rem(i, 4)
            mp = lax.rem(i + 3, 4)
            mn = lax.rem(i + 2, 4)

            @pl.when(i >= 1)
            def _wait_prev_scatter():
                pltpu.make_async_copy(
                    hrow.at[q], accum.at[didx.at[mp]], ssem).wait()

            @pl.when(i + 1 < NCHUNK)
            def _issue_next():
                pltpu.async_copy(h_hbm.at[sidx.at[q]], hrow.at[q], hsem)

            pltpu.make_async_copy(
                h_hbm.at[sidx.at[p]], hrow.at[p], hsem).wait()

            @pl.when(i + 2 < NCHUNK)
            def _load_next_idx():
                load_idx(i + 2, p, mn)

            @plsc.parallel_loop(0, C // 16, unroll=1)
            def grp_body(g):
                e0 = g * 16
                av = attrs[m, pl.ds(e0, 16)]
                for k in range(16):
                    a = av[k]
                    for j in range(8):
                        sl = pl.ds(j * 16, 16)
                        hrow[p, e0 + k, sl] = jnp.maximum(
                            hrow[p, e0 + k, sl] + embt[a, sl], 0.0)

            @pl.when(i + 2 < NCHUNK)
            def _widx():
                wait_idx()

            pltpu.async_copy(hrow.at[p], accum.at[didx.at[m]], ssem,
                             add=True)
            return carry

        lax.fori_loop(0, NCHUNK, chunk_body, 0)

        pltpu.make_async_copy(
            hrow.at[(NCHUNK - 1) % 2],
            accum.at[didx.at[(NCHUNK - 1) % 4]], ssem).wait()

        plsc.subcore_barrier()

        pltpu.sync_copy(accum.at[pl.ds(s * RPT, RPT)],
                        out_hbm.at[c, pl.ds(s * RPT, RPT)])

        @pl.when(s == NS - 1)
        def _copy_tail():
            pltpu.sync_copy(accum.at[pl.ds(NS * RPT, TAIL)],
                            out_hbm.at[c, pl.ds(NS * RPT, TAIL)])

    return agg(h, src, dst2, attr, emb)


BLK = 1000


def _enc_body(x_ref, w_ref, b_ref, o_ref):
    z = jnp.dot(x_ref[...], w_ref[...], preferred_element_type=jnp.float32)
    o_ref[...] = jnp.maximum(z + b_ref[...], 0.0)


def _encode(x, w, b):
    return pl.pallas_call(
        _enc_body,
        grid=(N // BLK,),
        in_specs=[
            pl.BlockSpec((BLK, D), lambda i: (i, 0)),
            pl.BlockSpec((D, D), lambda i: (0, 0)),
            pl.BlockSpec((1, D), lambda i: (0, 0)),
        ],
        out_specs=pl.BlockSpec((BLK, D), lambda i: (i, 0)),
        out_shape=jax.ShapeDtypeStruct((N, D), jnp.float32),
    )(x, w, b.reshape(1, D))


def _mlp_body(h_ref, p_ref, scale_ref, wa_ref, ba_ref, wb_ref, bb_ref,
              wc_ref, bc_ref, o_ref):
    z = h_ref[...] * scale_ref[...] + p_ref[0] + p_ref[1]
    z = jnp.dot(z, wa_ref[...], preferred_element_type=jnp.float32)
    z = jnp.maximum(z + ba_ref[...], 0.0)
    z = jnp.dot(z, wb_ref[...], preferred_element_type=jnp.float32)
    z = jnp.maximum(z + bb_ref[...], 0.0)
    z = jnp.dot(z, wc_ref[...], preferred_element_type=jnp.float32)
    o_ref[...] = jnp.maximum(z + bc_ref[...], 0.0)


def _mlp(h, partials, eps, wa, ba, wb, bb, wc, bc):
    scale = (1.0 + eps).reshape(1, 1)
    wspec = pl.BlockSpec((D, D), lambda i: (0, 0))
    bspec = pl.BlockSpec((1, D), lambda i: (0, 0))
    return pl.pallas_call(
        _mlp_body,
        grid=(N // BLK,),
        in_specs=[
            pl.BlockSpec((BLK, D), lambda i: (i, 0)),
            pl.BlockSpec((NC, BLK, D), lambda i: (0, i, 0)),
            pl.BlockSpec((1, 1), lambda i: (0, 0)),
            wspec, bspec, wspec, bspec, wspec, bspec,
        ],
        out_specs=pl.BlockSpec((BLK, D), lambda i: (i, 0)),
        out_shape=jax.ShapeDtypeStruct((N, D), jnp.float32),
    )(h, partials, scale, wa, ba.reshape(1, D), wb, bb.reshape(1, D),
      wc, bc.reshape(1, D))


def kernel(x, edge_index, edge_attr, W_enc, b_enc, emb, eps1,
           W1a, b1a, W1b, b1b, W1c, b1c, eps2, W2a, b2a, W2b, b2b, W2c, b2c):
    src = edge_index[0]
    dst2 = edge_index[1]

    h = _encode(x, W_enc, b_enc)
    p1 = _sc_aggregate(h, src, dst2, edge_attr, emb)
    h = _mlp(h, p1, eps1, W1a, b1a, W1b, b1b, W1c, b1c)
    p2 = _sc_aggregate(h, src, dst2, edge_attr, emb)
    h = _mlp(h, p2, eps2, W2a, b2a, W2b, b2b, W2c, b2c)
    return h


# speedup vs baseline: 1.0837x; 1.0837x over previous
import functools

import jax
import jax.numpy as jnp
from jax import lax
from jax.experimental import pallas as pl
from jax.experimental.pallas import tpu as pltpu
from jax.experimental.pallas import tpu_sc as plsc

N = 10000
E = 320000
D = 128

NC = 2
NS = 16
NW = NC * NS

EPT = E // NW
C = 80
NCHUNK = EPT // C
RPT = 624
TAIL = N - NS * RPT


def _sc_aggregate(h, src, dst2, attr, emb):
    mesh = plsc.VectorSubcoreMesh(core_axis_name="c", subcore_axis_name="s")

    @functools.partial(
        pl.kernel,
        mesh=mesh,
        out_type=jax.ShapeDtypeStruct((NC, N, D), jnp.float32),
        scratch_types=[
            pltpu.VMEM((2, C), jnp.int32),
            pltpu.VMEM((4, C), jnp.int32),
            pltpu.VMEM((4, C), jnp.int32),
            pltpu.VMEM((8, D), jnp.float32),
            pltpu.VMEM((2, C, D), jnp.float32),
            pltpu.VMEM_SHARED((N, D), jnp.float32),
            pltpu.SemaphoreType.DMA,
            pltpu.SemaphoreType.DMA,
            pltpu.SemaphoreType.DMA,
        ],
    )
    def agg(h_hbm, src_hbm, dst_hbm, attr_hbm, emb_hbm, out_hbm,
            sidx, didx, attrs, embt, hrow, accum,
            isem, hsem, ssem):
        c = lax.axis_index("c")
        s = lax.axis_index("s")
        base = (c * NS + s) * EPT

        pltpu.sync_copy(emb_hbm, embt)

        zero = jnp.zeros((16,), jnp.float32)

        def zb_body(i, carry):
            for b in range(2):
                for j in range(8):
                    hrow[b, i, pl.ds(j * 16, 16)] = zero
            return carry

        lax.fori_loop(0, C, zb_body, 0)
        for k in range(RPT // C):
            pltpu.sync_copy(hrow.at[0],
                            accum.at[pl.ds(s * RPT + k * C, C)])
        pltpu.sync_copy(hrow.at[0, pl.ds(0, RPT - (RPT // C) * C)],
                        accum.at[pl.ds(s * RPT + (RPT // C) * C,
                                       RPT - (RPT // C) * C)])

        @pl.when(s == NS - 1)
        def _zero_tail():
            pltpu.sync_copy(hrow.at[0, pl.ds(0, TAIL)],
                            accum.at[pl.ds(NS * RPT, TAIL)])

        def load_idx(chunk, ib, db):
            off = base + chunk * C
            pltpu.async_copy(src_hbm.at[pl.ds(off, C)], sidx.at[ib], isem)
            pltpu.async_copy(attr_hbm.at[pl.ds(off, C)], attrs.at[db], isem)
            pltpu.async_copy(dst_hbm.at[pl.ds(off, C)], didx.at[db], isem)

        def wait_idx():
            for _ in range(2):
                pltpu.make_async_copy(
                    src_hbm.at[pl.ds(0, C)], sidx.at[0], isem).wait()
            pltpu.make_async_copy(
                attr_hbm.at[pl.ds(0, C)], attrs.at[0], isem).wait()

        load_idx(0, 0, 0)
        load_idx(1, 1, 1)
        wait_idx()
        wait_idx()
        pltpu.async_copy(h_hbm.at[sidx.at[0]], hrow.at[0], hsem)

        plsc.subcore_barrier()

        def chunk_body(i, carry):
            p = lax.rem(i, 2)
            q = lax.rem(i + 1, 2)
            m = lax.rem(i, 4)
            mp = lax.rem(i + 3, 4)
            mn = lax.rem(i + 2, 4)

            @pl.when(i >= 1)
            def _wait_prev_scatter():
                pltpu.make_async_copy(
                    hrow.at[q], accum.at[didx.at[mp]], ssem).wait()

            @pl.when(i + 1 < NCHUNK)
            def _issue_next():
                pltpu.async_copy(h_hbm.at[sidx.at[q]], hrow.at[q], hsem)

            pltpu.make_async_copy(
                h_hbm.at[sidx.at[p]], hrow.at[p], hsem).wait()

            @pl.when(i + 2 < NCHUNK)
            def _load_next_idx():
                load_idx(i + 2, p, mn)

            @plsc.parallel_loop(0, C // 16, unroll=1)
            def grp_body(g):
                e0 = g * 16
                av = attrs[m, pl.ds(e0, 16)]
                for k in range(16):
                    a = av[k]
                    for j in range(8):
                        sl = pl.ds(j * 16, 16)
                        hrow[p, e0 + k, sl] = jnp.maximum(
                            hrow[p, e0 + k, sl] + embt[a, sl], 0.0)

            pltpu.async_copy(hrow.at[p], accum.at[didx.at[m]], ssem,
                             add=True)

            @pl.when(i + 2 < NCHUNK)
            def _widx():
                wait_idx()
            return carry

        lax.fori_loop(0, NCHUNK, chunk_body, 0)

        pltpu.make_async_copy(
            hrow.at[(NCHUNK - 1) % 2],
            accum.at[didx.at[(NCHUNK - 1) % 4]], ssem).wait()

        plsc.subcore_barrier()

        pltpu.sync_copy(accum.at[pl.ds(s * RPT, RPT)],
                        out_hbm.at[c, pl.ds(s * RPT, RPT)])

        @pl.when(s == NS - 1)
        def _copy_tail():
            pltpu.sync_copy(accum.at[pl.ds(NS * RPT, TAIL)],
                            out_hbm.at[c, pl.ds(NS * RPT, TAIL)])

    return agg(h, src, dst2, attr, emb)


BLK = 1000


def _enc_body(x_ref, w_ref, b_ref, o_ref):
    z = jnp.dot(x_ref[...], w_ref[...], preferred_element_type=jnp.float32)
    o_ref[...] = jnp.maximum(z + b_ref[...], 0.0)


def _encode(x, w, b):
    return pl.pallas_call(
        _enc_body,
        grid=(N // BLK,),
        in_specs=[
            pl.BlockSpec((BLK, D), lambda i: (i, 0)),
            pl.BlockSpec((D, D), lambda i: (0, 0)),
            pl.BlockSpec((1, D), lambda i: (0, 0)),
        ],
        out_specs=pl.BlockSpec((BLK, D), lambda i: (i, 0)),
        out_shape=jax.ShapeDtypeStruct((N, D), jnp.float32),
    )(x, w, b.reshape(1, D))


def _mlp_body(h_ref, p_ref, scale_ref, wa_ref, ba_ref, wb_ref, bb_ref,
              wc_ref, bc_ref, o_ref):
    z = h_ref[...] * scale_ref[...] + p_ref[0] + p_ref[1]
    z = jnp.dot(z, wa_ref[...], preferred_element_type=jnp.float32)
    z = jnp.maximum(z + ba_ref[...], 0.0)
    z = jnp.dot(z, wb_ref[...], preferred_element_type=jnp.float32)
    z = jnp.maximum(z + bb_ref[...], 0.0)
    z = jnp.dot(z, wc_ref[...], preferred_element_type=jnp.float32)
    o_ref[...] = jnp.maximum(z + bc_ref[...], 0.0)


def _mlp(h, partials, eps, wa, ba, wb, bb, wc, bc):
    scale = (1.0 + eps).reshape(1, 1)
    wspec = pl.BlockSpec((D, D), lambda i: (0, 0))
    bspec = pl.BlockSpec((1, D), lambda i: (0, 0))
    return pl.pallas_call(
        _mlp_body,
        grid=(N // BLK,),
        in_specs=[
            pl.BlockSpec((BLK, D), lambda i: (i, 0)),
            pl.BlockSpec((NC, BLK, D), lambda i: (0, i, 0)),
            pl.BlockSpec((1, 1), lambda i: (0, 0)),
            wspec, bspec, wspec, bspec, wspec, bspec,
        ],
        out_specs=pl.BlockSpec((BLK, D), lambda i: (i, 0)),
        out_shape=jax.ShapeDtypeStruct((N, D), jnp.float32),
    )(h, partials, scale, wa, ba.reshape(1, D), wb, bb.reshape(1, D),
      wc, bc.reshape(1, D))


def kernel(x, edge_index, edge_attr, W_enc, b_enc, emb, eps1,
           W1a, b1a, W1b, b1b, W1c, b1c, eps2, W2a, b2a, W2b, b2b, W2c, b2c):
    src = edge_index[0]
    dst2 = edge_index[1]

    h = _encode(x, W_enc, b_enc)
    p1 = _sc_aggregate(h, src, dst2, edge_attr, emb)
    h = _mlp(h, p1, eps1, W1a, b1a, W1b, b1b, W1c, b1c)
    p2 = _sc_aggregate(h, src, dst2, edge_attr, emb)
    h = _mlp(h, p2, eps2, W2a, b2a, W2b, b2b, W2c, b2c)
    return h
